# baseline (speedup 1.0000x reference)
import functools

import jax
import jax.numpy as jnp
from jax.experimental import pallas as pl

N = 10000
D = 256
K = 100
BM = 256


def _embed_body(f_ref, w1_ref, b1_ref, w2_ref, b2_ref, x_ref):
    f = f_ref[...]
    h = jax.lax.dot_general(f, w1_ref[...], (((1,), (1,)), ((), ())),
                            preferred_element_type=jnp.float32)
    h = jnp.maximum(h + b1_ref[...], 0.0)
    e = jax.lax.dot_general(h, w2_ref[...], (((1,), (1,)), ((), ())),
                            preferred_element_type=jnp.float32)
    e = e + b2_ref[...]
    nrm = jnp.sqrt(jnp.sum(e * e, axis=1, keepdims=True))
    x_ref[...] = e / jnp.maximum(nrm, 1e-12)


def _embed(features, W1, b1, W2, b2):
    n_pad = ((N + BM - 1) // BM) * BM
    f = jnp.pad(features, ((0, n_pad - N), (0, 0)))
    grid = (n_pad // BM,)
    x = pl.pallas_call(
        _embed_body,
        grid=grid,
        in_specs=[
            pl.BlockSpec((BM, D), lambda i: (i, 0)),
            pl.BlockSpec((D, D), lambda i: (0, 0)),
            pl.BlockSpec((1, D), lambda i: (0, 0)),
            pl.BlockSpec((D, D), lambda i: (0, 0)),
            pl.BlockSpec((1, D), lambda i: (0, 0)),
        ],
        out_specs=pl.BlockSpec((BM, D), lambda i: (i, 0)),
        out_shape=jax.ShapeDtypeStruct((n_pad, D), jnp.float32),
    )(f, W1, b1.reshape(1, D), W2, b2.reshape(1, D))
    return x[:N]


def kernel(features, W1, b1, W2, b2):
    X = _embed(features, W1, b1, W2, b2)
    sim = X @ X.T
    vals, inds = jax.lax.top_k(sim, K + 1)
    rows = jnp.repeat(jnp.arange(N, dtype=jnp.int32), K + 1)
    cols = inds.reshape(-1)
    values = vals.reshape(-1)
    norm_row = jnp.sum(vals, axis=1)
    norm_col = jnp.zeros((N,), dtype=values.dtype).at[cols].add(values)
    norm = norm_row + norm_col
    values = values * (norm[rows] ** -0.5) * (norm[cols] ** -0.5)
    edge_index = jnp.stack([rows, cols.astype(jnp.int32)], axis=0)
    return edge_index, values


# speedup vs baseline: 3.1487x; 3.1487x over previous
import functools

import jax
import jax.numpy as jnp
from jax import lax
from jax.experimental import pallas as pl
from jax.experimental.pallas import tpu as pltpu
from jax.experimental.pallas import tpu_sc as plsc

N = 10000
D = 256
K = 100
KP1 = K + 1
NV = N // 16
OUTW = 112
CAP = 128
BM = 256
NPAD = ((N + BM - 1) // BM) * BM
NTILES = 32
INT_MIN = -(2 ** 31)


def _embed_body(f_ref, w1_ref, b1_ref, w2_ref, b2_ref, x_ref):
    f = f_ref[...]
    h = lax.dot_general(f, w1_ref[...], (((1,), (1,)), ((), ())),
                        preferred_element_type=jnp.float32)
    h = jnp.maximum(h + b1_ref[...], 0.0)
    e = lax.dot_general(h, w2_ref[...], (((1,), (1,)), ((), ())),
                        preferred_element_type=jnp.float32)
    e = e + b2_ref[...]
    nrm = jnp.sqrt(jnp.sum(e * e, axis=1, keepdims=True))
    x_ref[...] = e / jnp.maximum(nrm, 1e-12)


def _embed(features, W1, b1, W2, b2):
    f = jnp.pad(features, ((0, NPAD - N), (0, 0)))
    return pl.pallas_call(
        _embed_body,
        grid=(NPAD // BM,),
        in_specs=[
            pl.BlockSpec((BM, D), lambda i: (i, 0)),
            pl.BlockSpec((D, D), lambda i: (0, 0)),
            pl.BlockSpec((1, D), lambda i: (0, 0)),
            pl.BlockSpec((D, D), lambda i: (0, 0)),
            pl.BlockSpec((1, D), lambda i: (0, 0)),
        ],
        out_specs=pl.BlockSpec((BM, D), lambda i: (i, 0)),
        out_shape=jax.ShapeDtypeStruct((NPAD, D), jnp.float32),
    )(f, W1, b1.reshape(1, D), W2, b2.reshape(1, D))


def _simkeys_body(xb_ref, x_ref, out_ref):
    s = lax.dot_general(xb_ref[...], x_ref[...], (((1,), (1,)), ((), ())),
                        preferred_element_type=jnp.float32)
    u = lax.bitcast_convert_type(s, jnp.int32)
    out_ref[...] = u ^ (lax.shift_right_arithmetic(u, 31) & 0x7FFFFFFF)


def _simkeys(xp, x):
    return pl.pallas_call(
        _simkeys_body,
        grid=(NPAD // BM,),
        in_specs=[
            pl.BlockSpec((BM, D), lambda i: (i, 0)),
            pl.BlockSpec((N, D), lambda i: (0, 0)),
        ],
        out_specs=pl.BlockSpec((BM, N), lambda i: (i, 0)),
        out_shape=jax.ShapeDtypeStruct((NPAD, N), jnp.int32),
    )(xp, x)


def _vsort16(k, v):
    return plsc.sort_key_val(k, v, descending=True)


def _finish(ks, vs):
    n = len(ks)
    if n == 1:
        k, v = _vsort16(ks[0], vs[0])
        return [k], [v]
    h = n // 2
    hk, hv, lk, lv = [], [], [], []
    for i in range(h):
        m = ks[i] >= ks[i + h]
        hk.append(jnp.where(m, ks[i], ks[i + h]))
        hv.append(jnp.where(m, vs[i], vs[i + h]))
        lk.append(jnp.where(m, ks[i + h], ks[i]))
        lv.append(jnp.where(m, vs[i + h], vs[i]))
    k1, v1 = _finish(hk, hv)
    k2, v2 = _finish(lk, lv)
    return k1 + k2, v1 + v2


def _merge(ak, av, bk, bv):
    n = len(ak)
    rbk = [lax.rev(x, (0,)) for x in reversed(bk)]
    rbv = [lax.rev(x, (0,)) for x in reversed(bv)]
    hk, hv, lk, lv = [], [], [], []
    for i in range(n):
        m = ak[i] >= rbk[i]
        hk.append(jnp.where(m, ak[i], rbk[i]))
        hv.append(jnp.where(m, av[i], rbv[i]))
        lk.append(jnp.where(m, rbk[i], ak[i]))
        lv.append(jnp.where(m, rbv[i], av[i]))
    k1, v1 = _finish(hk, hv)
    k2, v2 = _finish(lk, lv)
    return k1 + k2, v1 + v2


def _sort_desc(ks, vs):
    runs = []
    for k, v in zip(ks, vs):
        sk, sv = _vsort16(k, v)
        runs.append(([sk], [sv]))
    while len(runs) > 1:
        nxt = []
        for i in range(0, len(runs), 2):
            ak, av = runs[i]
            bk, bv = runs[i + 1]
            nxt.append(_merge(ak, av, bk, bv))
        runs = nxt
    return runs[0]


def _topk_sc(keys):
    mesh = plsc.VectorSubcoreMesh(core_axis_name="c", subcore_axis_name="s")

    @functools.partial(
        pl.kernel,
        mesh=mesh,
        compiler_params=pltpu.CompilerParams(needs_layout_passes=False),
        out_type=(
            jax.ShapeDtypeStruct((N, OUTW), jnp.float32),
            jax.ShapeDtypeStruct((N, OUTW), jnp.int32),
        ),
        scratch_types=[
            pltpu.VMEM((N,), jnp.int32),
            pltpu.VMEM((4096,), jnp.int32),
            pltpu.VMEM((256,), jnp.int32),
            pltpu.VMEM((CAP,), jnp.int32),
            pltpu.VMEM((CAP,), jnp.int32),
            pltpu.VMEM((OUTW,), jnp.float32),
            pltpu.VMEM((OUTW,), jnp.int32),
        ],
    )
    def topk(keys_hbm, vals_hbm, inds_hbm, row_v, hist_v, coarse_v,
             candk_v, candi_v, vst_v, ist_v):
        lane = jnp.arange(16, dtype=jnp.int32)
        ones16 = jnp.ones((16,), jnp.int32)
        zeros16 = jnp.zeros((16,), jnp.int32)
        wid = lax.axis_index("s") * 2 + lax.axis_index("c")
        nrows = jnp.where(wid < N - (N // NTILES) * NTILES,
                          N // NTILES + 1, N // NTILES)

        def hist_cnt(b):
            return jnp.sum(plsc.load_gather(hist_v, [b * 16 + lane]))

        def zero_hist(i, carry):
            plsc.store_scatter(hist_v, [i * 16 + lane], zeros16)
            return carry

        def row_body(j, carry):
            r = wid + NTILES * j
            pltpu.sync_copy(keys_hbm.at[r], row_v)
            lax.fori_loop(0, 256, zero_hist, 0)

            def h1(i, mx):
                kk = plsc.load_gather(row_v, [i * 16 + lane])
                b = lax.shift_right_arithmetic(kk, 24) + 128
                plsc.addupdate_scatter(hist_v, [b * 16 + lane], ones16)
                return jnp.maximum(mx, kk)

            mx = lax.fori_loop(0, NV, h1,
                               jnp.full((16,), INT_MIN, jnp.int32))
            maxk = jnp.max(mx)
            bstart = lax.shift_right_arithmetic(maxk, 24) + 128

            def scond(st):
                b, cum, cnt = st
                return cum + cnt < KP1

            def sbody(st):
                b, cum, cnt = st
                return (b - 1, cum + cnt, hist_cnt(b - 1))

            b1, g0, cnt1 = lax.while_loop(
                scond, sbody, (bstart, 0, hist_cnt(bstart)))
            prefix0 = b1 - 128
            cge0 = g0 + cnt1

            def rcond(st):
                _, sp, _, cge = st
                return (cge > CAP) & (sp > 0)

            def rbody(st):
                prefix, sp, g, _ = st
                lax.fori_loop(0, 256, zero_hist, 0)

                def zero_coarse(i, carry):
                    plsc.store_scatter(coarse_v, [i * 16 + lane], zeros16)
                    return carry

                lax.fori_loop(0, 16, zero_coarse, 0)
                spm8 = sp - 8

                def h2(i, carry):
                    kk = plsc.load_gather(row_v, [i * 16 + lane])
                    m = lax.shift_right_arithmetic(kk, sp) == prefix
                    b8 = lax.shift_right_arithmetic(kk, spm8) & 0xFF
                    plsc.addupdate_scatter(
                        hist_v, [b8 * 16 + lane], ones16, mask=m)
                    plsc.addupdate_scatter(
                        coarse_v, [(b8 >> 4) * 16 + lane], ones16, mask=m)
                    return carry

                lax.fori_loop(0, NV, h2, 0)
                need = KP1 - g

                def coarse_cnt(c):
                    return jnp.sum(plsc.load_gather(coarse_v,
                                                    [c * 16 + lane]))

                def ccond(st):
                    c, cum, cnt = st
                    return cum + cnt < need

                def cbody(st):
                    c, cum, cnt = st
                    return (c - 1, cum + cnt, coarse_cnt(c - 1))

                c1, cumc, _ = lax.while_loop(
                    ccond, cbody, (15, 0, coarse_cnt(15)))
                need2 = need - cumc
                f0 = c1 * 16 + 15

                def fcond(st):
                    b, cum, cnt = st
                    return cum + cnt < need2

                def fbody(st):
                    b, cum, cnt = st
                    return (b - 1, cum + cnt, hist_cnt(b - 1))

                b2, cumf, cntf = lax.while_loop(
                    fcond, fbody, (f0, 0, hist_cnt(f0)))
                gn = g + cumc + cumf
                return ((prefix << 8) | b2, spm8, gn, gn + cntf)

            prefix, sp, _, _ = lax.while_loop(
                rcond, rbody, (prefix0, 24, g0, cge0))

            for v in range(CAP // 16):
                plsc.store_scatter(candk_v, [v * 16 + lane],
                                   jnp.full((16,), INT_MIN, jnp.int32))
                plsc.store_scatter(candi_v, [v * 16 + lane], zeros16)

            def collect(i, off):
                kk = plsc.load_gather(row_v, [i * 16 + lane])
                m = lax.shift_right_arithmetic(kk, sp) >= prefix
                c = plsc.cumsum(m.astype(jnp.int32))
                pos = off + c - 1
                ok = m & (pos < CAP)
                plsc.store_scatter(candk_v, [pos], kk, mask=ok)
                plsc.store_scatter(candi_v, [pos], i * 16 + lane, mask=ok)
                return off + jnp.max(c)

            lax.fori_loop(0, NV, collect, 0)

            ks = [plsc.load_gather(candk_v, [v * 16 + lane])
                  for v in range(CAP // 16)]
            vs = [plsc.load_gather(candi_v, [v * 16 + lane])
                  for v in range(CAP // 16)]
            sk, sv = _sort_desc(ks, vs)
            for v in range(OUTW // 16):
                valid = (v * 16 + lane) < KP1
                kkk = sk[v]
                u = kkk ^ (lax.shift_right_arithmetic(kkk, 31) & 0x7FFFFFFF)
                f = lax.bitcast_convert_type(u, jnp.float32)
                vst_v[pl.ds(v * 16, 16)] = jnp.where(valid, f, 0.0)
                ist_v[pl.ds(v * 16, 16)] = jnp.where(valid, sv[v], 0)
            pltpu.sync_copy(vst_v, vals_hbm.at[r])
            pltpu.sync_copy(ist_v, inds_hbm.at[r])
            return carry

        lax.fori_loop(0, nrows, row_body, 0)

    return topk(keys)


def kernel(features, W1, b1, W2, b2):
    xp = _embed(features, W1, b1, W2, b2)
    x = xp[:N]
    keys = _simkeys(xp, x)
    vals_pad, inds_pad = _topk_sc(keys)
    vals = vals_pad[:, :KP1]
    inds = inds_pad[:, :KP1]
    rows = jnp.repeat(jnp.arange(N, dtype=jnp.int32), KP1)
    cols = inds.reshape(-1)
    values = vals.reshape(-1)
    norm_row = jnp.sum(vals, axis=1)
    norm_col = jnp.zeros((N,), dtype=values.dtype).at[cols].add(values)
    norm = norm_row + norm_col
    values = values * (norm[rows] ** -0.5) * (norm[cols] ** -0.5)
    edge_index = jnp.stack([rows, cols], axis=0)
    return edge_index, values


# speedup vs baseline: 8.5183x; 2.7053x over previous
import functools

import jax
import jax.numpy as jnp
from jax import lax
from jax.experimental import pallas as pl
from jax.experimental.pallas import tpu as pltpu
from jax.experimental.pallas import tpu_sc as plsc

N = 10000
D = 256
K = 100
KP1 = K + 1
NV = N // 16
OUTW = 112
CAP = 128
BM = 256
NPAD = ((N + BM - 1) // BM) * BM
NTILES = 32
INT_MIN = -(2 ** 31)


def _embed_body(f_ref, w1_ref, b1_ref, w2_ref, b2_ref, x_ref):
    f = f_ref[...]
    h = lax.dot_general(f, w1_ref[...], (((1,), (1,)), ((), ())),
                        preferred_element_type=jnp.float32)
    h = jnp.maximum(h + b1_ref[...], 0.0)
    e = lax.dot_general(h, w2_ref[...], (((1,), (1,)), ((), ())),
                        preferred_element_type=jnp.float32)
    e = e + b2_ref[...]
    nrm = jnp.sqrt(jnp.sum(e * e, axis=1, keepdims=True))
    x_ref[...] = e / jnp.maximum(nrm, 1e-12)


def _embed(features, W1, b1, W2, b2):
    f = jnp.pad(features, ((0, NPAD - N), (0, 0)))
    return pl.pallas_call(
        _embed_body,
        grid=(NPAD // BM,),
        in_specs=[
            pl.BlockSpec((BM, D), lambda i: (i, 0)),
            pl.BlockSpec((D, D), lambda i: (0, 0)),
            pl.BlockSpec((1, D), lambda i: (0, 0)),
            pl.BlockSpec((D, D), lambda i: (0, 0)),
            pl.BlockSpec((1, D), lambda i: (0, 0)),
        ],
        out_specs=pl.BlockSpec((BM, D), lambda i: (i, 0)),
        out_shape=jax.ShapeDtypeStruct((NPAD, D), jnp.float32),
    )(f, W1, b1.reshape(1, D), W2, b2.reshape(1, D))


def _simkeys_body(xb_ref, x_ref, out_ref):
    s = lax.dot_general(xb_ref[...], x_ref[...], (((1,), (1,)), ((), ())),
                        preferred_element_type=jnp.float32)
    u = lax.bitcast_convert_type(s, jnp.int32)
    out_ref[...] = u ^ (lax.shift_right_arithmetic(u, 31) & 0x7FFFFFFF)


def _simkeys(xp, x):
    return pl.pallas_call(
        _simkeys_body,
        grid=(NPAD // BM,),
        in_specs=[
            pl.BlockSpec((BM, D), lambda i: (i, 0)),
            pl.BlockSpec((N, D), lambda i: (0, 0)),
        ],
        out_specs=pl.BlockSpec((BM, N), lambda i: (i, 0)),
        out_shape=jax.ShapeDtypeStruct((NPAD, N), jnp.int32),
    )(xp, x)


def _vsort16(k, v):
    return plsc.sort_key_val(k, v, descending=True)


def _finish(ks, vs):
    n = len(ks)
    if n == 1:
        k, v = _vsort16(ks[0], vs[0])
        return [k], [v]
    h = n // 2
    hk, hv, lk, lv = [], [], [], []
    for i in range(h):
        m = ks[i] >= ks[i + h]
        hk.append(jnp.where(m, ks[i], ks[i + h]))
        hv.append(jnp.where(m, vs[i], vs[i + h]))
        lk.append(jnp.where(m, ks[i + h], ks[i]))
        lv.append(jnp.where(m, vs[i + h], vs[i]))
    k1, v1 = _finish(hk, hv)
    k2, v2 = _finish(lk, lv)
    return k1 + k2, v1 + v2


def _merge(ak, av, bk, bv):
    n = len(ak)
    rbk = [lax.rev(x, (0,)) for x in reversed(bk)]
    rbv = [lax.rev(x, (0,)) for x in reversed(bv)]
    hk, hv, lk, lv = [], [], [], []
    for i in range(n):
        m = ak[i] >= rbk[i]
        hk.append(jnp.where(m, ak[i], rbk[i]))
        hv.append(jnp.where(m, av[i], rbv[i]))
        lk.append(jnp.where(m, rbk[i], ak[i]))
        lv.append(jnp.where(m, rbv[i], av[i]))
    k1, v1 = _finish(hk, hv)
    k2, v2 = _finish(lk, lv)
    return k1 + k2, v1 + v2


def _sort_desc(ks, vs):
    runs = []
    for k, v in zip(ks, vs):
        sk, sv = _vsort16(k, v)
        runs.append(([sk], [sv]))
    while len(runs) > 1:
        nxt = []
        for i in range(0, len(runs), 2):
            ak, av = runs[i]
            bk, bv = runs[i + 1]
            nxt.append(_merge(ak, av, bk, bv))
        runs = nxt
    return runs[0]


def _topk_sc(keys):
    mesh = plsc.VectorSubcoreMesh(core_axis_name="c", subcore_axis_name="s")

    @functools.partial(
        pl.kernel,
        mesh=mesh,
        compiler_params=pltpu.CompilerParams(needs_layout_passes=False),
        out_type=(
            jax.ShapeDtypeStruct((N, OUTW), jnp.float32),
            jax.ShapeDtypeStruct((N, OUTW), jnp.int32),
        ),
        scratch_types=[
            pltpu.VMEM((N,), jnp.int32),
            pltpu.VMEM((4096,), jnp.int32),
            pltpu.VMEM((256,), jnp.int32),
            pltpu.VMEM((CAP,), jnp.int32),
            pltpu.VMEM((CAP,), jnp.int32),
            pltpu.VMEM((OUTW,), jnp.float32),
            pltpu.VMEM((OUTW,), jnp.int32),
        ],
    )
    def topk(keys_hbm, vals_hbm, inds_hbm, row_v, hist_v, coarse_v,
             candk_v, candi_v, vst_v, ist_v):
        lane = jnp.arange(16, dtype=jnp.int32)
        ones16 = jnp.ones((16,), jnp.int32)
        zeros16 = jnp.zeros((16,), jnp.int32)
        wid = lax.axis_index("s") * 2 + lax.axis_index("c")
        nrows = jnp.where(wid < N - (N // NTILES) * NTILES,
                          N // NTILES + 1, N // NTILES)

        def hist_cnt(b):
            return jnp.sum(plsc.load_gather(hist_v, [b * 16 + lane]))

        def zero_hist(i, carry):
            plsc.store_scatter(hist_v, [i * 16 + lane], zeros16)
            return carry

        def row_body(j, carry):
            r = wid + NTILES * j
            pltpu.sync_copy(keys_hbm.at[r], row_v)
            lax.fori_loop(0, 256, zero_hist, 0)

            def h1(i, mx):
                kk = plsc.load_gather(row_v, [i * 16 + lane])
                b = lax.shift_right_arithmetic(kk, 24) + 128
                plsc.addupdate_scatter(hist_v, [b * 16 + lane], ones16)
                return jnp.maximum(mx, kk)

            mx = lax.fori_loop(0, NV, h1,
                               jnp.full((16,), INT_MIN, jnp.int32))
            maxk = jnp.max(mx)
            bstart = lax.shift_right_arithmetic(maxk, 24) + 128

            def scond(st):
                b, cum, cnt = st
                return cum + cnt < KP1

            def sbody(st):
                b, cum, cnt = st
                return (b - 1, cum + cnt, hist_cnt(b - 1))

            b1, g0, cnt1 = lax.while_loop(
                scond, sbody, (bstart, 0, hist_cnt(bstart)))
            prefix0 = b1 - 128
            cge0 = g0 + cnt1

            def rcond(st):
                _, sp, _, cge = st
                return (cge > CAP) & (sp > 0)

            def rbody(st):
                prefix, sp, g, _ = st
                lax.fori_loop(0, 256, zero_hist, 0)

                def zero_coarse(i, carry):
                    plsc.store_scatter(coarse_v, [i * 16 + lane], zeros16)
                    return carry

                lax.fori_loop(0, 16, zero_coarse, 0)
                spm8 = sp - 8

                def h2(i, carry):
                    kk = plsc.load_gather(row_v, [i * 16 + lane])
                    m = lax.shift_right_arithmetic(kk, sp) == prefix
                    b8 = lax.shift_right_arithmetic(kk, spm8) & 0xFF
                    plsc.addupdate_scatter(
                        hist_v, [b8 * 16 + lane], ones16, mask=m)
                    plsc.addupdate_scatter(
                        coarse_v, [(b8 >> 4) * 16 + lane], ones16, mask=m)
                    return carry

                lax.fori_loop(0, NV, h2, 0)
                need = KP1 - g

                def coarse_cnt(c):
                    return jnp.sum(plsc.load_gather(coarse_v,
                                                    [c * 16 + lane]))

                def ccond(st):
                    c, cum, cnt = st
                    return cum + cnt < need

                def cbody(st):
                    c, cum, cnt = st
                    return (c - 1, cum + cnt, coarse_cnt(c - 1))

                c1, cumc, _ = lax.while_loop(
                    ccond, cbody, (15, 0, coarse_cnt(15)))
                need2 = need - cumc
                f0 = c1 * 16 + 15

                def fcond(st):
                    b, cum, cnt = st
                    return cum + cnt < need2

                def fbody(st):
                    b, cum, cnt = st
                    return (b - 1, cum + cnt, hist_cnt(b - 1))

                b2, cumf, cntf = lax.while_loop(
                    fcond, fbody, (f0, 0, hist_cnt(f0)))
                gn = g + cumc + cumf
                return ((prefix << 8) | b2, spm8, gn, gn + cntf)

            prefix, sp, _, _ = lax.while_loop(
                rcond, rbody, (prefix0, 24, g0, cge0))

            for v in range(CAP // 16):
                plsc.store_scatter(candk_v, [v * 16 + lane],
                                   jnp.full((16,), INT_MIN, jnp.int32))
                plsc.store_scatter(candi_v, [v * 16 + lane], zeros16)

            def collect(i, off):
                kk = plsc.load_gather(row_v, [i * 16 + lane])
                m = lax.shift_right_arithmetic(kk, sp) >= prefix
                c = plsc.cumsum(m.astype(jnp.int32))
                pos = off + c - 1
                ok = m & (pos < CAP)
                plsc.store_scatter(candk_v, [pos], kk, mask=ok)
                plsc.store_scatter(candi_v, [pos], i * 16 + lane, mask=ok)
                return off + jnp.max(c)

            lax.fori_loop(0, NV, collect, 0)

            ks = [plsc.load_gather(candk_v, [v * 16 + lane])
                  for v in range(CAP // 16)]
            vs = [plsc.load_gather(candi_v, [v * 16 + lane])
                  for v in range(CAP // 16)]
            sk, sv = _sort_desc(ks, vs)
            for v in range(OUTW // 16):
                valid = (v * 16 + lane) < KP1
                kkk = sk[v]
                u = kkk ^ (lax.shift_right_arithmetic(kkk, 31) & 0x7FFFFFFF)
                f = lax.bitcast_convert_type(u, jnp.float32)
                vst_v[pl.ds(v * 16, 16)] = jnp.where(valid, f, 0.0)
                ist_v[pl.ds(v * 16, 16)] = jnp.where(valid, sv[v], 0)
            pltpu.sync_copy(vst_v, vals_hbm.at[r])
            pltpu.sync_copy(ist_v, inds_hbm.at[r])
            return carry

        lax.fori_loop(0, nrows, row_body, 0)

    return topk(keys)


NROWP = 10016
CHUNK = 313 * OUTW


def _tile_start(wid):
    return 312 * wid + jnp.minimum(wid, 16)


def _colsum_sc(vals_flat, inds_flat):
    mesh = plsc.VectorSubcoreMesh(core_axis_name="c", subcore_axis_name="s")

    @functools.partial(
        pl.kernel,
        mesh=mesh,
        compiler_params=pltpu.CompilerParams(needs_layout_passes=False),
        out_type=jax.ShapeDtypeStruct((NTILES, N), jnp.float32),
        scratch_types=[
            pltpu.VMEM((CHUNK,), jnp.float32),
            pltpu.VMEM((CHUNK,), jnp.int32),
            pltpu.VMEM((N,), jnp.float32),
        ],
    )
    def colsum(vals_hbm, inds_hbm, out_hbm, vbuf, ibuf, hist_v):
        lane = jnp.arange(16, dtype=jnp.int32)
        zerosf = jnp.zeros((16,), jnp.float32)
        wid = lax.axis_index("s") * 2 + lax.axis_index("c")
        start = _tile_start(wid)
        nrows = jnp.minimum(313, N - start)
        pltpu.sync_copy(vals_hbm.at[pl.ds(start * OUTW, CHUNK)], vbuf)
        pltpu.sync_copy(inds_hbm.at[pl.ds(start * OUTW, CHUNK)], ibuf)

        def zero_hist(i, carry):
            plsc.store_scatter(hist_v, [i * 16 + lane], zerosf)
            return carry

        lax.fori_loop(0, N // 16, zero_hist, 0)

        def row_body(j, carry):
            base = j * OUTW
            for t in range(7):
                ix = plsc.load_gather(ibuf, [base + t * 16 + lane])
                v = plsc.load_gather(vbuf, [base + t * 16 + lane])
                if t == 6:
                    m = lane < (KP1 - 96)
                    plsc.addupdate_scatter(hist_v, [ix], v, mask=m)
                else:
                    plsc.addupdate_scatter(hist_v, [ix], v)
            return carry

        lax.fori_loop(0, nrows, row_body, 0)
        pltpu.sync_copy(hist_v, out_hbm.at[wid])

    return colsum(vals_flat, inds_flat)


def _invnorm_body(vt_ref, cp_ref, out_ref):
    norm = jnp.sum(vt_ref[...], axis=0) + jnp.sum(cp_ref[...], axis=0)
    out_ref[...] = lax.rsqrt(norm).reshape(1, N)


def _invnorm(vals_t, colpart):
    return pl.pallas_call(
        _invnorm_body,
        out_shape=jax.ShapeDtypeStruct((1, N), jnp.float32),
    )(vals_t, colpart)


def _scale_sc(vals_flat, inds_flat, inv):
    mesh = plsc.VectorSubcoreMesh(core_axis_name="c", subcore_axis_name="s")

    @functools.partial(
        pl.kernel,
        mesh=mesh,
        compiler_params=pltpu.CompilerParams(needs_layout_passes=False),
        out_type=jax.ShapeDtypeStruct((NROWP * OUTW,), jnp.float32),
        scratch_types=[
            pltpu.VMEM((CHUNK,), jnp.float32),
            pltpu.VMEM((CHUNK,), jnp.int32),
            pltpu.VMEM((N,), jnp.float32),
        ],
    )
    def scale(vals_hbm, inds_hbm, inv_hbm, out_hbm, vbuf, ibuf, inv_v):
        lane = jnp.arange(16, dtype=jnp.int32)
        wid = lax.axis_index("s") * 2 + lax.axis_index("c")
        start = _tile_start(wid)
        nrows = jnp.minimum(313, N - start)
        pltpu.sync_copy(vals_hbm.at[pl.ds(start * OUTW, CHUNK)], vbuf)
        pltpu.sync_copy(inds_hbm.at[pl.ds(start * OUTW, CHUNK)], ibuf)
        pltpu.sync_copy(inv_hbm, inv_v)

        def row_body(j, carry):
            base = j * OUTW
            r = start + j
            fr = plsc.load_gather(inv_v, [jnp.zeros((16,), jnp.int32) + r])
            for t in range(7):
                ix = plsc.load_gather(ibuf, [base + t * 16 + lane])
                v = plsc.load_gather(vbuf, [base + t * 16 + lane])
                fc = plsc.load_gather(inv_v, [ix])
                plsc.store_scatter(vbuf, [base + t * 16 + lane], v * fr * fc)
            return carry

        lax.fori_loop(0, nrows, row_body, 0)
        pltpu.sync_copy(vbuf, out_hbm.at[pl.ds(start * OUTW, CHUNK)])

    return scale(vals_flat, inds_flat, inv)


def kernel(features, W1, b1, W2, b2):
    xp = _embed(features, W1, b1, W2, b2)
    x = xp[:N]
    keys = _simkeys(xp, x)
    vals_pad, inds_pad = _topk_sc(keys)
    vp = jnp.pad(vals_pad, ((0, NROWP - N), (0, 0)))
    ip = jnp.pad(inds_pad, ((0, NROWP - N), (0, 0)))
    vals_flat = vp.reshape(-1)
    inds_flat = ip.reshape(-1)
    colpart = _colsum_sc(vals_flat, inds_flat)
    inv = _invnorm(vals_pad.T, colpart).reshape(-1)
    scaled = _scale_sc(vals_flat, inds_flat, inv)
    values = scaled.reshape(NROWP, OUTW)[:N, :KP1].reshape(-1)
    inds = inds_pad[:, :KP1]
    rows = jnp.repeat(jnp.arange(N, dtype=jnp.int32), KP1)
    cols = inds.reshape(-1)
    edge_index = jnp.stack([rows, cols], axis=0)
    return edge_index, values


# speedup vs baseline: 10.6782x; 1.2536x over previous
import functools

import jax
import jax.numpy as jnp
from jax import lax
from jax.experimental import pallas as pl
from jax.experimental.pallas import tpu as pltpu
from jax.experimental.pallas import tpu_sc as plsc

N = 10000
D = 256
K = 100
KP1 = K + 1
NV = N // 16
OUTW = 112
CAP = 128
BM = 256
NPAD = ((N + BM - 1) // BM) * BM
NTILES = 32
INT_MIN = -(2 ** 31)


def _embed_body(f_ref, w1_ref, b1_ref, w2_ref, b2_ref, x_ref):
    f = f_ref[...]
    h = lax.dot_general(f, w1_ref[...], (((1,), (1,)), ((), ())),
                        preferred_element_type=jnp.float32)
    h = jnp.maximum(h + b1_ref[...], 0.0)
    e = lax.dot_general(h, w2_ref[...], (((1,), (1,)), ((), ())),
                        preferred_element_type=jnp.float32)
    e = e + b2_ref[...]
    nrm = jnp.sqrt(jnp.sum(e * e, axis=1, keepdims=True))
    x_ref[...] = e / jnp.maximum(nrm, 1e-12)


def _embed(features, W1, b1, W2, b2):
    f = jnp.pad(features, ((0, NPAD - N), (0, 0)))
    return pl.pallas_call(
        _embed_body,
        grid=(NPAD // BM,),
        in_specs=[
            pl.BlockSpec((BM, D), lambda i: (i, 0)),
            pl.BlockSpec((D, D), lambda i: (0, 0)),
            pl.BlockSpec((1, D), lambda i: (0, 0)),
            pl.BlockSpec((D, D), lambda i: (0, 0)),
            pl.BlockSpec((1, D), lambda i: (0, 0)),
        ],
        out_specs=pl.BlockSpec((BM, D), lambda i: (i, 0)),
        out_shape=jax.ShapeDtypeStruct((NPAD, D), jnp.float32),
    )(f, W1, b1.reshape(1, D), W2, b2.reshape(1, D))


def _simkeys_body(xb_ref, x_ref, out_ref):
    s = lax.dot_general(xb_ref[...], x_ref[...], (((1,), (1,)), ((), ())),
                        preferred_element_type=jnp.float32)
    u = lax.bitcast_convert_type(s, jnp.int32)
    out_ref[...] = u ^ (lax.shift_right_arithmetic(u, 31) & 0x7FFFFFFF)


def _simkeys(xp, x):
    return pl.pallas_call(
        _simkeys_body,
        grid=(NPAD // BM,),
        in_specs=[
            pl.BlockSpec((BM, D), lambda i: (i, 0)),
            pl.BlockSpec((N, D), lambda i: (0, 0)),
        ],
        out_specs=pl.BlockSpec((BM, N), lambda i: (i, 0)),
        out_shape=jax.ShapeDtypeStruct((NPAD, N), jnp.int32),
    )(xp, x)


def _vsort16(k, v):
    return plsc.sort_key_val(k, v, descending=True)


def _finish(ks, vs):
    n = len(ks)
    if n == 1:
        k, v = _vsort16(ks[0], vs[0])
        return [k], [v]
    h = n // 2
    hk, hv, lk, lv = [], [], [], []
    for i in range(h):
        m = ks[i] >= ks[i + h]
        hk.append(jnp.where(m, ks[i], ks[i + h]))
        hv.append(jnp.where(m, vs[i], vs[i + h]))
        lk.append(jnp.where(m, ks[i + h], ks[i]))
        lv.append(jnp.where(m, vs[i + h], vs[i]))
    k1, v1 = _finish(hk, hv)
    k2, v2 = _finish(lk, lv)
    return k1 + k2, v1 + v2


def _merge(ak, av, bk, bv):
    n = len(ak)
    rbk = [lax.rev(x, (0,)) for x in reversed(bk)]
    rbv = [lax.rev(x, (0,)) for x in reversed(bv)]
    hk, hv, lk, lv = [], [], [], []
    for i in range(n):
        m = ak[i] >= rbk[i]
        hk.append(jnp.where(m, ak[i], rbk[i]))
        hv.append(jnp.where(m, av[i], rbv[i]))
        lk.append(jnp.where(m, rbk[i], ak[i]))
        lv.append(jnp.where(m, rbv[i], av[i]))
    k1, v1 = _finish(hk, hv)
    k2, v2 = _finish(lk, lv)
    return k1 + k2, v1 + v2


def _sort_desc(ks, vs):
    runs = []
    for k, v in zip(ks, vs):
        sk, sv = _vsort16(k, v)
        runs.append(([sk], [sv]))
    while len(runs) > 1:
        nxt = []
        for i in range(0, len(runs), 2):
            ak, av = runs[i]
            bk, bv = runs[i + 1]
            nxt.append(_merge(ak, av, bk, bv))
        runs = nxt
    return runs[0]


def _prune_top(ak, av, bk, bv):
    n = len(ak)
    rbk = [lax.rev(x, (0,)) for x in reversed(bk)]
    rbv = [lax.rev(x, (0,)) for x in reversed(bv)]
    hk, hv = [], []
    for i in range(n):
        m = ak[i] >= rbk[i]
        hk.append(jnp.where(m, ak[i], rbk[i]))
        hv.append(jnp.where(m, av[i], rbv[i]))
    return _finish(hk, hv)


CAND = 256
NVM = (NV - 1) // 4


def _topk_sc(keys):
    mesh = plsc.VectorSubcoreMesh(core_axis_name="c", subcore_axis_name="s")

    @functools.partial(
        pl.kernel,
        mesh=mesh,
        compiler_params=pltpu.CompilerParams(needs_layout_passes=False),
        out_type=(
            jax.ShapeDtypeStruct((N, OUTW), jnp.float32),
            jax.ShapeDtypeStruct((N, OUTW), jnp.int32),
        ),
        scratch_types=[
            pltpu.VMEM((2, N), jnp.int32),
            pltpu.VMEM((4096,), jnp.int32),
            pltpu.VMEM((CAND,), jnp.int32),
            pltpu.VMEM((CAND,), jnp.int32),
            pltpu.VMEM((2, OUTW), jnp.float32),
            pltpu.VMEM((2, OUTW), jnp.int32),
            pltpu.SemaphoreType.DMA,
            pltpu.SemaphoreType.DMA,
            pltpu.SemaphoreType.DMA,
        ],
    )
    def topk(keys_hbm, vals_hbm, inds_hbm, row_v, hist_v, candk_v, candi_v,
             vst_v, ist_v, insem, vsem, isem):
        lane = jnp.arange(16, dtype=jnp.int32)
        ones16 = jnp.ones((16,), jnp.int32)
        zeros16 = jnp.zeros((16,), jnp.int32)
        wid = lax.axis_index("s") * 2 + lax.axis_index("c")
        start = 312 * wid + jnp.minimum(wid, 16)
        nrows = jnp.minimum(313, N - start)

        def hist_cnt(b):
            return jnp.sum(plsc.load_gather(hist_v, [b * 16 + lane]))

        def zero_hist_dyn(i, carry):
            plsc.store_scatter(hist_v, [i * 16 + lane], zeros16)
            return carry

        pltpu.make_async_copy(
            keys_hbm.at[start], row_v.at[0], insem).start()

        def row_body(j, carry):
            r = start + j
            p = j & 1
            psplat = jnp.zeros((16,), jnp.int32) + p
            pltpu.make_async_copy(
                keys_hbm.at[start], row_v.at[0], insem).wait()

            @pl.when(j + 1 < nrows)
            def _():
                pltpu.make_async_copy(
                    keys_hbm.at[r + 1], row_v.at[1 - p], insem).start()

            def ld(i):
                return plsc.load_gather(row_v, [psplat, i * 16 + lane])

            def mm_body(j4, mm):
                mn, mx = mm
                for k4 in range(4):
                    kk = ld(j4 * 4 + k4)
                    mn = jnp.minimum(mn, kk)
                    mx = jnp.maximum(mx, kk)
                return (mn, mx)

            mn0 = jnp.full((16,), 2 ** 31 - 1, jnp.int32)
            mx0 = jnp.full((16,), INT_MIN, jnp.int32)
            mnv, mxv = lax.fori_loop(0, NVM, mm_body, (mn0, mx0))
            kk_last = ld(NV - 1)
            lo = jnp.min(jnp.minimum(mnv, kk_last))
            mx = jnp.max(jnp.maximum(mxv, kk_last))

            delta = lax.bitcast_convert_type(mx - lo, jnp.uint32)
            s = jnp.uint32(0)
            for b in (16, 8, 4, 2, 1):
                s = jnp.where((delta >> (s + b)) != 0, s + b, s)
            s = jnp.maximum(s, 7) - 7
            s_i = s.astype(jnp.int32)

            for v in range(256):
                hist_v[pl.ds(v * 16, 16)] = zeros16

            def h_body(j4, carry):
                for k4 in range(4):
                    kk = ld(j4 * 4 + k4)
                    kd = lax.bitcast_convert_type(kk - lo, jnp.uint32)
                    b = (kd >> s).astype(jnp.int32)
                    plsc.addupdate_scatter(
                        hist_v, [(b << 4) | lane], ones16)
                return carry

            lax.fori_loop(0, NVM, h_body, 0)
            kd_last = lax.bitcast_convert_type(kk_last - lo, jnp.uint32)
            b_last = (kd_last >> s).astype(jnp.int32)
            plsc.addupdate_scatter(hist_v, [(b_last << 4) | lane], ones16)

            bmax = lax.shift_right_logical(
                lax.bitcast_convert_type(mx - lo, jnp.uint32), s
            ).astype(jnp.int32)

            def scond(st):
                b, cum, cnt = st
                return cum + cnt < KP1

            def sbody(st):
                b, cum, cnt = st
                return (b - 1, cum + cnt, hist_cnt(b - 1))

            b1, g0, cnt1 = lax.while_loop(
                scond, sbody, (bmax, 0, hist_cnt(bmax)))
            cge0 = g0 + cnt1

            def rcond(st):
                lo_w, sw, g, b1w, cge = st
                return (cge > CAND) & (sw > 0)

            def rbody(st):
                lo_w, sw, g, b1w, cge = st
                wlo = lo_w + lax.shift_left(b1w, sw)
                sw2 = jnp.maximum(sw, 8) - 8
                width = lax.bitcast_convert_type(
                    lax.shift_left(1, sw), jnp.uint32)
                lax.fori_loop(0, 256, zero_hist_dyn, 0)
                swu = sw2.astype(jnp.uint32)

                def h2(i, carry):
                    kk = ld(i)
                    du = lax.bitcast_convert_type(kk - wlo, jnp.uint32)
                    m = du < width
                    b = (du >> swu).astype(jnp.int32)
                    plsc.addupdate_scatter(
                        hist_v, [(b << 4) | lane], ones16, mask=m)
                    return carry

                lax.fori_loop(0, NV, h2, 0)
                need = KP1 - g
                topb = lax.shift_right_logical(
                    lax.shift_left(1, sw) - 1, sw2)

                def fcond(st2):
                    b, cum, cnt = st2
                    return cum + cnt < need

                def fbody(st2):
                    b, cum, cnt = st2
                    return (b - 1, cum + cnt, hist_cnt(b - 1))

                b2, cum2, cnt2 = lax.while_loop(
                    fcond, fbody, (topb, 0, hist_cnt(topb)))
                gn = g + cum2
                return (wlo, sw2, gn, b2, gn + cnt2)

            lo_f, s_f, _, b1_f, _ = lax.while_loop(
                rcond, rbody, (lo, s_i, g0, b1, cge0))
            thr = lo_f + lax.shift_left(b1_f, s_f)

            for v in range(CAND // 16):
                candk_v[pl.ds(v * 16, 16)] = jnp.full(
                    (16,), INT_MIN, jnp.int32)
                candi_v[pl.ds(v * 16, 16)] = zeros16

            def c_body(j4, off):
                for k4 in range(4):
                    i = j4 * 4 + k4
                    kk = ld(i)
                    m = kk >= thr
                    c = plsc.cumsum(m.astype(jnp.int32))
                    pos = off + c - 1
                    ok = m & (pos < CAND)
                    plsc.store_scatter(candk_v, [pos], kk, mask=ok)
                    plsc.store_scatter(
                        candi_v, [pos], i * 16 + lane, mask=ok)
                    off = off + plsc.all_reduce_population_count(m)
                return off

            off = lax.fori_loop(0, NVM, c_body, zeros16)
            m = kk_last >= thr
            c = plsc.cumsum(m.astype(jnp.int32))
            pos = off + c - 1
            ok = m & (pos < CAND)
            plsc.store_scatter(candk_v, [pos], kk_last, mask=ok)
            plsc.store_scatter(candi_v, [pos], (NV - 1) * 16 + lane, mask=ok)

            ks = [candk_v[pl.ds(v * 16, 16)] for v in range(CAND // 16)]
            vs = [candi_v[pl.ds(v * 16, 16)] for v in range(CAND // 16)]
            ak, av = _sort_desc(ks[:8], vs[:8])
            bk, bv = _sort_desc(ks[8:], vs[8:])
            sk, sv = _prune_top(ak, av, bk, bv)

            @pl.when(j >= 2)
            def _():
                pltpu.make_async_copy(
                    vst_v.at[0], vals_hbm.at[start], vsem).wait()
                pltpu.make_async_copy(
                    ist_v.at[0], inds_hbm.at[start], isem).wait()

            for v in range(OUTW // 16):
                valid = (v * 16 + lane) < KP1
                kkk = sk[v]
                u = kkk ^ (lax.shift_right_arithmetic(kkk, 31) & 0x7FFFFFFF)
                f = lax.bitcast_convert_type(u, jnp.float32)
                plsc.store_scatter(
                    vst_v, [psplat, v * 16 + lane], jnp.where(valid, f, 0.0))
                plsc.store_scatter(
                    ist_v, [psplat, v * 16 + lane],
                    jnp.where(valid, sv[v], 0))
            pltpu.make_async_copy(
                vst_v.at[p], vals_hbm.at[r], vsem).start()
            pltpu.make_async_copy(
                ist_v.at[p], inds_hbm.at[r], isem).start()
            return carry

        lax.fori_loop(0, nrows, row_body, 0)
        for _ in range(2):
            pltpu.make_async_copy(
                vst_v.at[0], vals_hbm.at[start], vsem).wait()
            pltpu.make_async_copy(
                ist_v.at[0], inds_hbm.at[start], isem).wait()

    return topk(keys)


NROWP = 10016
CHUNK = 313 * OUTW


def _tile_start(wid):
    return 312 * wid + jnp.minimum(wid, 16)


def _colsum_sc(vals_flat, inds_flat):
    mesh = plsc.VectorSubcoreMesh(core_axis_name="c", subcore_axis_name="s")

    @functools.partial(
        pl.kernel,
        mesh=mesh,
        compiler_params=pltpu.CompilerParams(needs_layout_passes=False),
        out_type=jax.ShapeDtypeStruct((NTILES, N), jnp.float32),
        scratch_types=[
            pltpu.VMEM((CHUNK,), jnp.float32),
            pltpu.VMEM((CHUNK,), jnp.int32),
            pltpu.VMEM((N,), jnp.float32),
        ],
    )
    def colsum(vals_hbm, inds_hbm, out_hbm, vbuf, ibuf, hist_v):
        lane = jnp.arange(16, dtype=jnp.int32)
        zerosf = jnp.zeros((16,), jnp.float32)
        wid = lax.axis_index("s") * 2 + lax.axis_index("c")
        start = _tile_start(wid)
        nrows = jnp.minimum(313, N - start)
        pltpu.sync_copy(vals_hbm.at[pl.ds(start * OUTW, CHUNK)], vbuf)
        pltpu.sync_copy(inds_hbm.at[pl.ds(start * OUTW, CHUNK)], ibuf)

        def zero_hist(i, carry):
            plsc.store_scatter(hist_v, [i * 16 + lane], zerosf)
            return carry

        lax.fori_loop(0, N // 16, zero_hist, 0)

        def row_body(j, carry):
            base = j * OUTW
            for t in range(7):
                ix = plsc.load_gather(ibuf, [base + t * 16 + lane])
                v = plsc.load_gather(vbuf, [base + t * 16 + lane])
                if t == 6:
                    m = lane < (KP1 - 96)
                    plsc.addupdate_scatter(hist_v, [ix], v, mask=m)
                else:
                    plsc.addupdate_scatter(hist_v, [ix], v)
            return carry

        lax.fori_loop(0, nrows, row_body, 0)
        pltpu.sync_copy(hist_v, out_hbm.at[wid])

    return colsum(vals_flat, inds_flat)


def _invnorm_body(vt_ref, cp_ref, out_ref):
    norm = jnp.sum(vt_ref[...], axis=0) + jnp.sum(cp_ref[...], axis=0)
    out_ref[...] = lax.rsqrt(norm).reshape(1, N)


def _invnorm(vals_t, colpart):
    return pl.pallas_call(
        _invnorm_body,
        out_shape=jax.ShapeDtypeStruct((1, N), jnp.float32),
    )(vals_t, colpart)


def _scale_sc(vals_flat, inds_flat, inv):
    mesh = plsc.VectorSubcoreMesh(core_axis_name="c", subcore_axis_name="s")

    @functools.partial(
        pl.kernel,
        mesh=mesh,
        compiler_params=pltpu.CompilerParams(needs_layout_passes=False),
        out_type=jax.ShapeDtypeStruct((NROWP * OUTW,), jnp.float32),
        scratch_types=[
            pltpu.VMEM((CHUNK,), jnp.float32),
            pltpu.VMEM((CHUNK,), jnp.int32),
            pltpu.VMEM((N,), jnp.float32),
        ],
    )
    def scale(vals_hbm, inds_hbm, inv_hbm, out_hbm, vbuf, ibuf, inv_v):
        lane = jnp.arange(16, dtype=jnp.int32)
        wid = lax.axis_index("s") * 2 + lax.axis_index("c")
        start = _tile_start(wid)
        nrows = jnp.minimum(313, N - start)
        pltpu.sync_copy(vals_hbm.at[pl.ds(start * OUTW, CHUNK)], vbuf)
        pltpu.sync_copy(inds_hbm.at[pl.ds(start * OUTW, CHUNK)], ibuf)
        pltpu.sync_copy(inv_hbm, inv_v)

        def row_body(j, carry):
            base = j * OUTW
            r = start + j
            fr = plsc.load_gather(inv_v, [jnp.zeros((16,), jnp.int32) + r])
            for t in range(7):
                ix = plsc.load_gather(ibuf, [base + t * 16 + lane])
                v = plsc.load_gather(vbuf, [base + t * 16 + lane])
                fc = plsc.load_gather(inv_v, [ix])
                plsc.store_scatter(vbuf, [base + t * 16 + lane], v * fr * fc)
            return carry

        lax.fori_loop(0, nrows, row_body, 0)
        pltpu.sync_copy(vbuf, out_hbm.at[pl.ds(start * OUTW, CHUNK)])

    return scale(vals_flat, inds_flat, inv)


def kernel(features, W1, b1, W2, b2):
    xp = _embed(features, W1, b1, W2, b2)
    x = xp[:N]
    keys = _simkeys(xp, x)
    vals_pad, inds_pad = _topk_sc(keys)
    vp = jnp.pad(vals_pad, ((0, NROWP - N), (0, 0)))
    ip = jnp.pad(inds_pad, ((0, NROWP - N), (0, 0)))
    vals_flat = vp.reshape(-1)
    inds_flat = ip.reshape(-1)
    colpart = _colsum_sc(vals_flat, inds_flat)
    inv = _invnorm(vals_pad.T, colpart).reshape(-1)
    scaled = _scale_sc(vals_flat, inds_flat, inv)
    values = scaled.reshape(NROWP, OUTW)[:N, :KP1].reshape(-1)
    inds = inds_pad[:, :KP1]
    rows = jnp.repeat(jnp.arange(N, dtype=jnp.int32), KP1)
    cols = inds.reshape(-1)
    edge_index = jnp.stack([rows, cols], axis=0)
    return edge_index, values


# speedup vs baseline: 10.6915x; 1.0012x over previous
import functools

import jax
import jax.numpy as jnp
from jax import lax
from jax.experimental import pallas as pl
from jax.experimental.pallas import tpu as pltpu
from jax.experimental.pallas import tpu_sc as plsc

N = 10000
D = 256
K = 100
KP1 = K + 1
NV = N // 16
OUTW = 112
CAP = 128
BM = 256
NPAD = ((N + BM - 1) // BM) * BM
NTILES = 32
INT_MIN = -(2 ** 31)


def _embed_body(f_ref, w1_ref, b1_ref, w2_ref, b2_ref, x_ref):
    f = f_ref[...]
    h = lax.dot_general(f, w1_ref[...], (((1,), (1,)), ((), ())),
                        preferred_element_type=jnp.float32)
    h = jnp.maximum(h + b1_ref[...], 0.0)
    e = lax.dot_general(h, w2_ref[...], (((1,), (1,)), ((), ())),
                        preferred_element_type=jnp.float32)
    e = e + b2_ref[...]
    nrm = jnp.sqrt(jnp.sum(e * e, axis=1, keepdims=True))
    x_ref[...] = e / jnp.maximum(nrm, 1e-12)


def _embed(features, W1, b1, W2, b2):
    f = jnp.pad(features, ((0, NPAD - N), (0, 0)))
    return pl.pallas_call(
        _embed_body,
        grid=(NPAD // BM,),
        in_specs=[
            pl.BlockSpec((BM, D), lambda i: (i, 0)),
            pl.BlockSpec((D, D), lambda i: (0, 0)),
            pl.BlockSpec((1, D), lambda i: (0, 0)),
            pl.BlockSpec((D, D), lambda i: (0, 0)),
            pl.BlockSpec((1, D), lambda i: (0, 0)),
        ],
        out_specs=pl.BlockSpec((BM, D), lambda i: (i, 0)),
        out_shape=jax.ShapeDtypeStruct((NPAD, D), jnp.float32),
    )(f, W1, b1.reshape(1, D), W2, b2.reshape(1, D))


def _simkeys_body(xb_ref, x_ref, out_ref):
    s = lax.dot_general(xb_ref[...], x_ref[...], (((1,), (1,)), ((), ())),
                        preferred_element_type=jnp.float32)
    u = lax.bitcast_convert_type(s, jnp.int32)
    out_ref[...] = u ^ (lax.shift_right_arithmetic(u, 31) & 0x7FFFFFFF)


def _simkeys(xp, x):
    return pl.pallas_call(
        _simkeys_body,
        grid=(NPAD // BM,),
        in_specs=[
            pl.BlockSpec((BM, D), lambda i: (i, 0)),
            pl.BlockSpec((N, D), lambda i: (0, 0)),
        ],
        out_specs=pl.BlockSpec((BM, N), lambda i: (i, 0)),
        out_shape=jax.ShapeDtypeStruct((NPAD, N), jnp.int32),
    )(xp, x)


def _vsort16(k, v):
    return plsc.sort_key_val(k, v, descending=True)


def _finish(ks, vs):
    n = len(ks)
    if n == 1:
        k, v = _vsort16(ks[0], vs[0])
        return [k], [v]
    h = n // 2
    hk, hv, lk, lv = [], [], [], []
    for i in range(h):
        m = ks[i] >= ks[i + h]
        hk.append(jnp.where(m, ks[i], ks[i + h]))
        hv.append(jnp.where(m, vs[i], vs[i + h]))
        lk.append(jnp.where(m, ks[i + h], ks[i]))
        lv.append(jnp.where(m, vs[i + h], vs[i]))
    k1, v1 = _finish(hk, hv)
    k2, v2 = _finish(lk, lv)
    return k1 + k2, v1 + v2


def _merge(ak, av, bk, bv):
    n = len(ak)
    rbk = [lax.rev(x, (0,)) for x in reversed(bk)]
    rbv = [lax.rev(x, (0,)) for x in reversed(bv)]
    hk, hv, lk, lv = [], [], [], []
    for i in range(n):
        m = ak[i] >= rbk[i]
        hk.append(jnp.where(m, ak[i], rbk[i]))
        hv.append(jnp.where(m, av[i], rbv[i]))
        lk.append(jnp.where(m, rbk[i], ak[i]))
        lv.append(jnp.where(m, rbv[i], av[i]))
    k1, v1 = _finish(hk, hv)
    k2, v2 = _finish(lk, lv)
    return k1 + k2, v1 + v2


def _sort_desc(ks, vs):
    runs = []
    for k, v in zip(ks, vs):
        sk, sv = _vsort16(k, v)
        runs.append(([sk], [sv]))
    while len(runs) > 1:
        nxt = []
        for i in range(0, len(runs), 2):
            ak, av = runs[i]
            bk, bv = runs[i + 1]
            nxt.append(_merge(ak, av, bk, bv))
        runs = nxt
    return runs[0]


def _prune_top(ak, av, bk, bv):
    n = len(ak)
    rbk = [lax.rev(x, (0,)) for x in reversed(bk)]
    rbv = [lax.rev(x, (0,)) for x in reversed(bv)]
    hk, hv = [], []
    for i in range(n):
        m = ak[i] >= rbk[i]
        hk.append(jnp.where(m, ak[i], rbk[i]))
        hv.append(jnp.where(m, av[i], rbv[i]))
    return _finish(hk, hv)


CAND = 128
NVM = (NV - 1) // 4


def _topk_sc(keys):
    mesh = plsc.VectorSubcoreMesh(core_axis_name="c", subcore_axis_name="s")

    @functools.partial(
        pl.kernel,
        mesh=mesh,
        compiler_params=pltpu.CompilerParams(needs_layout_passes=False),
        out_type=(
            jax.ShapeDtypeStruct((N, OUTW), jnp.float32),
            jax.ShapeDtypeStruct((N, OUTW), jnp.int32),
        ),
        scratch_types=[
            pltpu.VMEM((2, N), jnp.int32),
            pltpu.VMEM((4096,), jnp.int32),
            pltpu.VMEM((CAND,), jnp.int32),
            pltpu.VMEM((CAND,), jnp.int32),
            pltpu.VMEM((2, OUTW), jnp.float32),
            pltpu.VMEM((2, OUTW), jnp.int32),
            pltpu.SemaphoreType.DMA,
            pltpu.SemaphoreType.DMA,
            pltpu.SemaphoreType.DMA,
        ],
    )
    def topk(keys_hbm, vals_hbm, inds_hbm, row_v, hist_v, candk_v, candi_v,
             vst_v, ist_v, insem, vsem, isem):
        lane = jnp.arange(16, dtype=jnp.int32)
        ones16 = jnp.ones((16,), jnp.int32)
        zeros16 = jnp.zeros((16,), jnp.int32)
        wid = lax.axis_index("s") * 2 + lax.axis_index("c")
        start = 312 * wid + jnp.minimum(wid, 16)
        nrows = jnp.minimum(313, N - start)

        def hist_cnt(b):
            return jnp.sum(plsc.load_gather(hist_v, [b * 16 + lane]))

        def zero_hist_dyn(i, carry):
            plsc.store_scatter(hist_v, [i * 16 + lane], zeros16)
            return carry

        pltpu.make_async_copy(
            keys_hbm.at[start], row_v.at[0], insem).start()

        def row_body(j, carry):
            r = start + j
            p = j & 1
            psplat = jnp.zeros((16,), jnp.int32) + p
            pltpu.make_async_copy(
                keys_hbm.at[start], row_v.at[0], insem).wait()

            @pl.when(j + 1 < nrows)
            def _():
                pltpu.make_async_copy(
                    keys_hbm.at[r + 1], row_v.at[1 - p], insem).start()

            def ld(i):
                return plsc.load_gather(row_v, [psplat, i * 16 + lane])

            def mm_body(j4, mm):
                mn, mx, mx2 = mm
                for k4 in range(4):
                    kk = ld(j4 * 4 + k4)
                    mn = jnp.minimum(mn, kk)
                    mx2 = jnp.maximum(mx2, jnp.minimum(mx, kk))
                    mx = jnp.maximum(mx, kk)
                return (mn, mx, mx2)

            mn0 = jnp.full((16,), 2 ** 31 - 1, jnp.int32)
            mx0 = jnp.full((16,), INT_MIN, jnp.int32)
            mnv, mxv, mx2v = lax.fori_loop(0, NVM, mm_body, (mn0, mx0, mx0))
            kk_last = ld(NV - 1)
            mnv = jnp.minimum(mnv, kk_last)
            mx2v = jnp.maximum(mx2v, jnp.minimum(mxv, kk_last))
            mxv = jnp.maximum(mxv, kk_last)
            lo = jnp.min(mnv)
            gmx = jnp.max(mxv)
            nmaxlane = plsc.all_reduce_population_count(mxv == gmx)
            masked = jnp.where(mxv == gmx, INT_MIN, mxv)
            sml = jnp.where(nmaxlane >= 2, gmx, jnp.max(masked))
            gmx2 = jnp.max(jnp.maximum(mx2v, sml))

            delta = lax.bitcast_convert_type(gmx2 - lo, jnp.uint32)
            s = jnp.uint32(0)
            for b in (16, 8, 4, 2, 1):
                s = jnp.where((delta >> (s + b)) != 0, s + b, s)
            s = jnp.maximum(s, 7) - 7
            s_i = s.astype(jnp.int32)

            for v in range(256):
                hist_v[pl.ds(v * 16, 16)] = zeros16

            def h_body(j4, carry):
                for k4 in range(4):
                    kk = ld(j4 * 4 + k4)
                    b = jnp.minimum(
                        lax.shift_right_arithmetic(kk - lo, s_i), 255)
                    plsc.addupdate_scatter(
                        hist_v, [(b << 4) | lane], ones16)
                return carry

            lax.fori_loop(0, NVM, h_body, 0)
            b_last = jnp.minimum(
                lax.shift_right_arithmetic(kk_last - lo, s_i), 255)
            plsc.addupdate_scatter(hist_v, [(b_last << 4) | lane], ones16)

            bstart = lax.shift_right_logical(delta, s).astype(jnp.int32)
            cum0 = jnp.where((gmx > gmx2) & (bstart < 255), 1, 0)

            def scond(st):
                b, cum, cnt = st
                return cum + cnt < KP1

            def sbody(st):
                b, cum, cnt = st
                return (b - 1, cum + cnt, hist_cnt(b - 1))

            b1, g0, cnt1 = lax.while_loop(
                scond, sbody, (bstart, cum0, hist_cnt(bstart)))
            cge0 = g0 + cnt1

            def rcond(st):
                lo_w, sw, g, b1w, cge = st
                return (cge > CAND) & (sw > 0)

            def rbody(st):
                lo_w, sw, g, b1w, cge = st
                wlo = lo_w + lax.shift_left(b1w, sw)
                width = lax.bitcast_convert_type(
                    lax.shift_left(1, sw), jnp.uint32)
                width = jnp.where(
                    b1w >= 255,
                    jnp.maximum(
                        width,
                        lax.bitcast_convert_type(gmx - wlo, jnp.uint32) + 1),
                    width)
                swu = jnp.uint32(0)
                for b in (16, 8, 4, 2, 1):
                    swu = jnp.where(
                        ((width - 1) >> (swu + b)) != 0, swu + b, swu)
                swu = jnp.maximum(swu, 7) - 7
                sw2 = swu.astype(jnp.int32)
                lax.fori_loop(0, 256, zero_hist_dyn, 0)

                def h2(i, carry):
                    kk = ld(i)
                    du = lax.bitcast_convert_type(kk - wlo, jnp.uint32)
                    m = du < width
                    b = (du >> swu).astype(jnp.int32)
                    plsc.addupdate_scatter(
                        hist_v, [(b << 4) | lane], ones16, mask=m)
                    return carry

                lax.fori_loop(0, NV, h2, 0)
                need = KP1 - g
                topb = ((width - 1) >> swu).astype(jnp.int32)

                def fcond(st2):
                    b, cum, cnt = st2
                    return cum + cnt < need

                def fbody(st2):
                    b, cum, cnt = st2
                    return (b - 1, cum + cnt, hist_cnt(b - 1))

                b2, cum2, cnt2 = lax.while_loop(
                    fcond, fbody, (topb, 0, hist_cnt(topb)))
                gn = g + cum2
                return (wlo, sw2, gn, b2, gn + cnt2)

            lo_f, s_f, _, b1_f, _ = lax.while_loop(
                rcond, rbody, (lo, s_i, g0, b1, cge0))
            thr = lo_f + lax.shift_left(b1_f, s_f)

            for v in range(CAND // 16):
                candk_v[pl.ds(v * 16, 16)] = jnp.full(
                    (16,), INT_MIN, jnp.int32)
                candi_v[pl.ds(v * 16, 16)] = zeros16

            def c_body(j4, off):
                for k4 in range(4):
                    i = j4 * 4 + k4
                    kk = ld(i)
                    m = kk >= thr
                    c = plsc.cumsum(m.astype(jnp.int32))
                    pos = off + c - 1
                    ok = m & (pos < CAND)
                    plsc.store_scatter(candk_v, [pos], kk, mask=ok)
                    plsc.store_scatter(
                        candi_v, [pos], i * 16 + lane, mask=ok)
                    off = off + plsc.all_reduce_population_count(m)
                return off

            off = lax.fori_loop(0, NVM, c_body, zeros16)
            m = kk_last >= thr
            c = plsc.cumsum(m.astype(jnp.int32))
            pos = off + c - 1
            ok = m & (pos < CAND)
            plsc.store_scatter(candk_v, [pos], kk_last, mask=ok)
            plsc.store_scatter(candi_v, [pos], (NV - 1) * 16 + lane, mask=ok)

            ks = [candk_v[pl.ds(v * 16, 16)] for v in range(CAND // 16)]
            vs = [candi_v[pl.ds(v * 16, 16)] for v in range(CAND // 16)]
            sk, sv = _sort_desc(ks, vs)

            @pl.when(j >= 2)
            def _():
                pltpu.make_async_copy(
                    vst_v.at[0], vals_hbm.at[start], vsem).wait()
                pltpu.make_async_copy(
                    ist_v.at[0], inds_hbm.at[start], isem).wait()

            for v in range(OUTW // 16):
                valid = (v * 16 + lane) < KP1
                kkk = sk[v]
                u = kkk ^ (lax.shift_right_arithmetic(kkk, 31) & 0x7FFFFFFF)
                f = lax.bitcast_convert_type(u, jnp.float32)
                plsc.store_scatter(
                    vst_v, [psplat, v * 16 + lane], jnp.where(valid, f, 0.0))
                plsc.store_scatter(
                    ist_v, [psplat, v * 16 + lane],
                    jnp.where(valid, sv[v], 0))
            pltpu.make_async_copy(
                vst_v.at[p], vals_hbm.at[r], vsem).start()
            pltpu.make_async_copy(
                ist_v.at[p], inds_hbm.at[r], isem).start()
            return carry

        lax.fori_loop(0, nrows, row_body, 0)
        for _ in range(2):
            pltpu.make_async_copy(
                vst_v.at[0], vals_hbm.at[start], vsem).wait()
            pltpu.make_async_copy(
                ist_v.at[0], inds_hbm.at[start], isem).wait()

    return topk(keys)


NROWP = 10016
CHUNK = 313 * OUTW


def _tile_start(wid):
    return 312 * wid + jnp.minimum(wid, 16)


def _colsum_sc(vals_flat, inds_flat):
    mesh = plsc.VectorSubcoreMesh(core_axis_name="c", subcore_axis_name="s")

    @functools.partial(
        pl.kernel,
        mesh=mesh,
        compiler_params=pltpu.CompilerParams(needs_layout_passes=False),
        out_type=jax.ShapeDtypeStruct((NTILES, N), jnp.float32),
        scratch_types=[
            pltpu.VMEM((CHUNK,), jnp.float32),
            pltpu.VMEM((CHUNK,), jnp.int32),
            pltpu.VMEM((N,), jnp.float32),
        ],
    )
    def colsum(vals_hbm, inds_hbm, out_hbm, vbuf, ibuf, hist_v):
        lane = jnp.arange(16, dtype=jnp.int32)
        zerosf = jnp.zeros((16,), jnp.float32)
        wid = lax.axis_index("s") * 2 + lax.axis_index("c")
        start = _tile_start(wid)
        nrows = jnp.minimum(313, N - start)
        pltpu.sync_copy(vals_hbm.at[pl.ds(start * OUTW, CHUNK)], vbuf)
        pltpu.sync_copy(inds_hbm.at[pl.ds(start * OUTW, CHUNK)], ibuf)

        def zero_hist(i, carry):
            plsc.store_scatter(hist_v, [i * 16 + lane], zerosf)
            return carry

        lax.fori_loop(0, N // 16, zero_hist, 0)

        def row_body(j, carry):
            base = j * OUTW
            for t in range(7):
                ix = plsc.load_gather(ibuf, [base + t * 16 + lane])
                v = plsc.load_gather(vbuf, [base + t * 16 + lane])
                if t == 6:
                    m = lane < (KP1 - 96)
                    plsc.addupdate_scatter(hist_v, [ix], v, mask=m)
                else:
                    plsc.addupdate_scatter(hist_v, [ix], v)
            return carry

        lax.fori_loop(0, nrows, row_body, 0)
        pltpu.sync_copy(hist_v, out_hbm.at[wid])

    return colsum(vals_flat, inds_flat)


def _invnorm_body(vt_ref, cp_ref, out_ref):
    norm = jnp.sum(vt_ref[...], axis=0) + jnp.sum(cp_ref[...], axis=0)
    out_ref[...] = lax.rsqrt(norm).reshape(1, N)


def _invnorm(vals_t, colpart):
    return pl.pallas_call(
        _invnorm_body,
        out_shape=jax.ShapeDtypeStruct((1, N), jnp.float32),
    )(vals_t, colpart)


def _scale_sc(vals_flat, inds_flat, inv):
    mesh = plsc.VectorSubcoreMesh(core_axis_name="c", subcore_axis_name="s")

    @functools.partial(
        pl.kernel,
        mesh=mesh,
        compiler_params=pltpu.CompilerParams(needs_layout_passes=False),
        out_type=jax.ShapeDtypeStruct((NROWP * OUTW,), jnp.float32),
        scratch_types=[
            pltpu.VMEM((CHUNK,), jnp.float32),
            pltpu.VMEM((CHUNK,), jnp.int32),
            pltpu.VMEM((N,), jnp.float32),
        ],
    )
    def scale(vals_hbm, inds_hbm, inv_hbm, out_hbm, vbuf, ibuf, inv_v):
        lane = jnp.arange(16, dtype=jnp.int32)
        wid = lax.axis_index("s") * 2 + lax.axis_index("c")
        start = _tile_start(wid)
        nrows = jnp.minimum(313, N - start)
        pltpu.sync_copy(vals_hbm.at[pl.ds(start * OUTW, CHUNK)], vbuf)
        pltpu.sync_copy(inds_hbm.at[pl.ds(start * OUTW, CHUNK)], ibuf)
        pltpu.sync_copy(inv_hbm, inv_v)

        def row_body(j, carry):
            base = j * OUTW
            r = start + j
            fr = plsc.load_gather(inv_v, [jnp.zeros((16,), jnp.int32) + r])
            for t in range(7):
                ix = plsc.load_gather(ibuf, [base + t * 16 + lane])
                v = plsc.load_gather(vbuf, [base + t * 16 + lane])
                fc = plsc.load_gather(inv_v, [ix])
                plsc.store_scatter(vbuf, [base + t * 16 + lane], v * fr * fc)
            return carry

        lax.fori_loop(0, nrows, row_body, 0)
        pltpu.sync_copy(vbuf, out_hbm.at[pl.ds(start * OUTW, CHUNK)])

    return scale(vals_flat, inds_flat, inv)


def kernel(features, W1, b1, W2, b2):
    xp = _embed(features, W1, b1, W2, b2)
    x = xp[:N]
    keys = _simkeys(xp, x)
    vals_pad, inds_pad = _topk_sc(keys)
    vp = jnp.pad(vals_pad, ((0, NROWP - N), (0, 0)))
    ip = jnp.pad(inds_pad, ((0, NROWP - N), (0, 0)))
    vals_flat = vp.reshape(-1)
    inds_flat = ip.reshape(-1)
    colpart = _colsum_sc(vals_flat, inds_flat)
    inv = _invnorm(vals_pad.T, colpart).reshape(-1)
    scaled = _scale_sc(vals_flat, inds_flat, inv)
    values = scaled.reshape(NROWP, OUTW)[:N, :KP1].reshape(-1)
    inds = inds_pad[:, :KP1]
    rows = jnp.repeat(jnp.arange(N, dtype=jnp.int32), KP1)
    cols = inds.reshape(-1)
    edge_index = jnp.stack([rows, cols], axis=0)
    return edge_index, values
